# baseline (speedup 1.0000x reference)
import functools

import jax
import jax.numpy as jnp
from jax import lax
from jax.experimental import pallas as pl
from jax.experimental.pallas import tpu as pltpu
from jax.experimental.pallas import tpu_sc as plsc

_B, _L, _V, _D = 4096, 50, 100000, 32
_NW = 32
_CH = 128


def _sc_body(feats, ws, out, idx_v, rows0, rows1, rows2,
             rows3, tile0, tile1, gsem0, gsem1, gsem2, gsem3, osem0,
             osem1):
    w = lax.axis_index("c") * 16 + lax.axis_index("s")

    lane = lax.iota(jnp.int32, 16)
    row_ids = [lane + g * 16 for g in range(8)]

    def transpose(rv, tv):
        def dblock(j, c):
            for dd in range(4):
                fm = (lane + (j * 4 + dd)) & 31
                for g in range(8):
                    vals = plsc.load_gather(rv, [row_ids[g], fm])
                    plsc.store_scatter(tv, [fm, row_ids[g]], vals)
            return c

        lax.fori_loop(0, 8, dblock, 0)

    rows = (rows0, rows1, rows2, rows3)
    gsems = (gsem0, gsem1, gsem2, gsem3)
    tiles = (tile0, tile1)
    osems = (osem0, osem1)

    for t in range(3):
        f_hbm = feats.at[t]
        w_hbm = ws.at[t]
        pltpu.sync_copy(f_hbm.at[:, pl.ds(w * _CH, _CH)], idx_v)
        for p in range(4):
            pltpu.async_copy(w_hbm.at[idx_v.at[p]], rows[p], gsems[p])

        def wait_gather(sem):
            pltpu.make_async_copy(w_hbm.at[pl.ds(0, _CH)], rows0,
                                  sem).wait()

        def drain_out(tv, sem):
            for j in range(4):
                pltpu.make_async_copy(w_hbm.at[pl.ds(0, 8)],
                                      tv.at[pl.ds(8 * j, 8)], sem).wait()

        def emit(tv, sem, l):
            for j in range(4):
                pltpu.async_copy(tv.at[pl.ds(8 * j, 8)],
                                 out.at[l, (4 * t + j) * _NW + w], sem)

        def phase(l, rows_c, tv, gsem, osem, need_drain):
            wait_gather(gsem)

            if need_drain is True:
                drain_out(tv, osem)
            else:
                @pl.when(need_drain)
                def _():
                    drain_out(tv, osem)

            transpose(rows_c, tv)

            @pl.when(l + 4 < _L)
            def _():
                pltpu.async_copy(w_hbm.at[idx_v.at[l + 4]], rows_c,
                                 gsem)

            emit(tv, osem, l)

        def body(k, carry):
            l0 = 4 * k
            for p in range(4):
                l = l0 + p
                need_drain = True if p >= 2 else (k > 0)

                @pl.when(l < _L)
                def _(p=p, l=l, need_drain=need_drain):
                    phase(l, rows[p], tiles[p % 2], gsems[p],
                          osems[p % 2], need_drain)
            return carry

        lax.fori_loop(0, (_L + 3) // 4, body, 0)
        drain_out(tile0, osem0)
        drain_out(tile1, osem1)

    return


_mesh = plsc.VectorSubcoreMesh(core_axis_name="c", subcore_axis_name="s")

_gather3 = functools.partial(
    pl.kernel,
    out_type=jax.ShapeDtypeStruct((_L, 12 * _NW, 8, _CH), jnp.float32),
    mesh=_mesh,
    scratch_types=[
        pltpu.VMEM((_L, _CH), jnp.int32),
        pltpu.VMEM((_CH, _D), jnp.float32),
        pltpu.VMEM((_CH, _D), jnp.float32),
        pltpu.VMEM((_CH, _D), jnp.float32),
        pltpu.VMEM((_CH, _D), jnp.float32),
        pltpu.VMEM((_D, _CH), jnp.float32),
        pltpu.VMEM((_D, _CH), jnp.float32),
        pltpu.SemaphoreType.DMA,
        pltpu.SemaphoreType.DMA,
        pltpu.SemaphoreType.DMA,
        pltpu.SemaphoreType.DMA,
        pltpu.SemaphoreType.DMA,
        pltpu.SemaphoreType.DMA,
    ],
    compiler_params=pltpu.CompilerParams(
        use_tc_tiling_on_sc=False, needs_layout_passes=False),
)(_sc_body)


def kernel(feat_a, feat_b, feat_c, W_a, W_b, W_c):
    feats = jnp.stack([feat_a.T, feat_b.T, feat_c.T])
    ws = jnp.stack([W_a, W_b, W_c])
    out = _gather3(feats, ws)
    r = out.reshape(_L, 12, _NW, 8, _CH)
    r = r.transpose(0, 1, 3, 2, 4)
    r = r.reshape(_L, 3 * _D, _B)
    return r.transpose(0, 2, 1)


# speedup vs baseline: 1.0977x; 1.0977x over previous
import functools

import jax
import jax.numpy as jnp
from jax import lax
from jax.experimental import pallas as pl
from jax.experimental.pallas import tpu as pltpu
from jax.experimental.pallas import tpu_sc as plsc

_B, _L, _V, _D = 4096, 50, 100000, 32
_NW = 32
_CH = 128


def _sc_body(fa, fb, fc, wa, wb, wc, out, idx_v, rows0, rows1, rows2,
             rows3, tile0, tile1, gsem0, gsem1, gsem2, gsem3, osem0,
             osem1):
    w = lax.axis_index("c") * 16 + lax.axis_index("s")

    lane = lax.iota(jnp.int32, 16)
    row_ids = [lane + g * 16 for g in range(8)]

    def transpose(rv, tv):
        def dblock(j, c):
            for dd in range(4):
                fm = (lane + (j * 4 + dd)) & 31
                for g in range(8):
                    vals = plsc.load_gather(rv, [row_ids[g], fm])
                    plsc.store_scatter(tv, [fm, row_ids[g]], vals)
            return c

        lax.fori_loop(0, 8, dblock, 0)

    rows = (rows0, rows1, rows2, rows3)
    gsems = (gsem0, gsem1, gsem2, gsem3)
    tiles = (tile0, tile1)
    osems = (osem0, osem1)

    for t, (f_hbm, w_hbm) in enumerate(((fa, wa), (fb, wb), (fc, wc))):
        pltpu.sync_copy(f_hbm.at[:, pl.ds(w * _CH, _CH)], idx_v)
        for p in range(4):
            pltpu.async_copy(w_hbm.at[idx_v.at[p]], rows[p], gsems[p])

        def wait_gather(sem):
            pltpu.make_async_copy(w_hbm.at[pl.ds(0, _CH)], rows0,
                                  sem).wait()

        def drain_out(tv, sem):
            for j in range(4):
                pltpu.make_async_copy(w_hbm.at[pl.ds(0, 8)],
                                      tv.at[pl.ds(8 * j, 8)], sem).wait()

        def emit(tv, sem, l):
            for j in range(4):
                pltpu.async_copy(tv.at[pl.ds(8 * j, 8)],
                                 out.at[l, (4 * t + j) * _NW + w], sem)

        def phase(l, rows_c, tv, gsem, osem, need_drain):
            wait_gather(gsem)

            if need_drain is True:
                drain_out(tv, osem)
            else:
                @pl.when(need_drain)
                def _():
                    drain_out(tv, osem)

            transpose(rows_c, tv)

            @pl.when(l + 4 < _L)
            def _():
                pltpu.async_copy(w_hbm.at[idx_v.at[l + 4]], rows_c,
                                 gsem)

            emit(tv, osem, l)

        def body(k, carry):
            l0 = 4 * k
            for p in range(4):
                l = l0 + p
                need_drain = True if p >= 2 else (k > 0)

                @pl.when(l < _L)
                def _(p=p, l=l, need_drain=need_drain):
                    phase(l, rows[p], tiles[p % 2], gsems[p],
                          osems[p % 2], need_drain)
            return carry

        lax.fori_loop(0, (_L + 3) // 4, body, 0)
        drain_out(tile0, osem0)
        drain_out(tile1, osem1)

    return


_mesh = plsc.VectorSubcoreMesh(core_axis_name="c", subcore_axis_name="s")

_gather3 = functools.partial(
    pl.kernel,
    out_type=jax.ShapeDtypeStruct((_L, 12 * _NW, 8, _CH), jnp.float32),
    mesh=_mesh,
    scratch_types=[
        pltpu.VMEM((_L, _CH), jnp.int32),
        pltpu.VMEM((_CH, _D), jnp.float32),
        pltpu.VMEM((_CH, _D), jnp.float32),
        pltpu.VMEM((_CH, _D), jnp.float32),
        pltpu.VMEM((_CH, _D), jnp.float32),
        pltpu.VMEM((_D, _CH), jnp.float32),
        pltpu.VMEM((_D, _CH), jnp.float32),
        pltpu.SemaphoreType.DMA,
        pltpu.SemaphoreType.DMA,
        pltpu.SemaphoreType.DMA,
        pltpu.SemaphoreType.DMA,
        pltpu.SemaphoreType.DMA,
        pltpu.SemaphoreType.DMA,
    ],
    compiler_params=pltpu.CompilerParams(
        use_tc_tiling_on_sc=False, needs_layout_passes=False),
)(_sc_body)


def kernel(feat_a, feat_b, feat_c, W_a, W_b, W_c):
    out = _gather3(feat_a.T, feat_b.T, feat_c.T, W_a, W_b, W_c)
    r = out.reshape(_L, 12, _NW, 8, _CH)
    r = r.transpose(0, 1, 3, 2, 4)
    r = r.reshape(_L, 3 * _D, _B)
    return r.transpose(0, 2, 1)
